# Initial kernel scaffold; baseline (speedup 1.0000x reference)
import jax
import jax.numpy as jnp
from jax.experimental import pallas as pl


def kernel(x, edge_index, edge_attr, batch, params):
    raise NotImplementedError("write your pallas kernel here")


# speedup vs baseline: 4.5920x; 4.5920x over previous
import jax
import jax.numpy as jnp
from jax import lax
from jax.experimental import pallas as pl
from jax.experimental.pallas import tpu as pltpu
from jax.experimental.pallas import tpu_sc as plsc

N = 10000
E = 320000
D_IN = 128
ED = 16
NSEG = 64
EPS = 1e-5

BLK = 128
NBLK = E // BLK
NC, NS = 2, 16
NW = NC * NS
NBP = 80
NPAD = NW * NBP
N2 = 10240
NSLOT = 8
DEPTH = 4
ROWS_T = N2 // NS

_f32 = jnp.float32
_mesh = plsc.VectorSubcoreMesh(core_axis_name="c", subcore_axis_name="s")
_sc_params = pltpu.CompilerParams(use_tc_tiling_on_sc=False)
_sds = jax.ShapeDtypeStruct


def _seg_sum_body(g_hbm, src_hbm, dst_hbm, z_hbm, s_out,
                  sidx, didx, rows, acc, gsem, ssem):
    cid = lax.axis_index("c")
    sid = lax.axis_index("s")
    w = sid * NC + cid

    pltpu.sync_copy(z_hbm.at[pl.ds(sid * ROWS_T, ROWS_T)],
                    acc.at[pl.ds(sid * ROWS_T, ROWS_T)])
    plsc.subcore_barrier()

    pltpu.sync_copy(src_hbm.at[w], sidx)
    pltpu.sync_copy(dst_hbm.at[w], didx)

    def fire_gather(j, s):
        pltpu.async_copy(g_hbm.at[sidx.at[j]], rows.at[s], gsem.at[s])

    for s in range(DEPTH):
        fire_gather(jnp.int32(s), s)

    def it(j, carry):
        s = lax.rem(j, NSLOT)
        pltpu.make_async_copy(g_hbm.at[sidx.at[j]], rows.at[s], gsem.at[s]).wait()
        pltpu.async_copy(rows.at[s], acc.at[didx.at[j]], ssem.at[s], add=True)
        jn = j + DEPTH
        sn = lax.rem(jn, NSLOT)

        @pl.when(jn < NBP)
        def _():
            jo = jn - NSLOT

            @pl.when(jo >= 0)
            def _():
                pltpu.make_async_copy(rows.at[sn], acc.at[didx.at[jo]],
                                      ssem.at[sn]).wait()
            fire_gather(jn, sn)
        return carry

    lax.fori_loop(0, NBP, it, 0)

    for s in range(NSLOT):
        pltpu.make_async_copy(rows.at[s], acc.at[didx.at[0]], ssem.at[s]).wait()

    plsc.subcore_barrier()
    pltpu.sync_copy(acc.at[pl.ds(sid * ROWS_T, ROWS_T)],
                    s_out.at[cid, pl.ds(sid * ROWS_T, ROWS_T)])


_sc_seg_sum = pl.kernel(
    _seg_sum_body,
    out_type=[_sds((NC, N2, 64), _f32)],
    mesh=_mesh,
    scratch_types=[
        pltpu.VMEM((NBP, BLK), jnp.int32),
        pltpu.VMEM((NBP, BLK), jnp.int32),
        pltpu.VMEM((NSLOT, BLK, 64), _f32),
        pltpu.VMEM_SHARED((N2, 64), _f32),
        pltpu.SemaphoreType.DMA((NSLOT,)),
        pltpu.SemaphoreType.DMA((NSLOT,)),
    ],
    compiler_params=_sc_params,
)


def _ea_body(ea_hbm, dst_hbm, z_hbm, ea_out, didx, rows, acc, gsem, ssem):
    cid = lax.axis_index("c")
    sid = lax.axis_index("s")
    w = sid * NC + cid

    pltpu.sync_copy(z_hbm.at[pl.ds(sid * ROWS_T, ROWS_T)],
                    acc.at[pl.ds(sid * ROWS_T, ROWS_T)])
    plsc.subcore_barrier()

    pltpu.sync_copy(dst_hbm.at[w], didx)

    def real(j):
        return w * NBP + j < NBLK

    def fire_gather(j, s):
        @pl.when(real(j))
        def _():
            pltpu.async_copy(ea_hbm.at[w * NBP + j], rows.at[s], gsem.at[s])

    for s in range(DEPTH):
        fire_gather(jnp.int32(s), s)

    def it(j, carry):
        s = lax.rem(j, NSLOT)

        @pl.when(real(j))
        def _():
            pltpu.make_async_copy(ea_hbm.at[w * NBP + j], rows.at[s],
                                  gsem.at[s]).wait()
            pltpu.async_copy(rows.at[s], acc.at[didx.at[j]], ssem.at[s],
                             add=True)
        jn = j + DEPTH
        sn = lax.rem(jn, NSLOT)

        @pl.when(jn < NBP)
        def _():
            jo = jn - NSLOT

            @pl.when(jnp.logical_and(jo >= 0, real(jo)))
            def _():
                pltpu.make_async_copy(rows.at[sn], acc.at[didx.at[jo]],
                                      ssem.at[sn]).wait()
            fire_gather(jn, sn)
        return carry

    lax.fori_loop(0, NBP, it, 0)

    for s in range(NSLOT):
        @pl.when(real(jnp.int32(NBP - NSLOT + s)))
        def _():
            pltpu.make_async_copy(rows.at[s], acc.at[didx.at[0]],
                                  ssem.at[s]).wait()

    plsc.subcore_barrier()
    pltpu.sync_copy(acc.at[pl.ds(sid * ROWS_T, ROWS_T)],
                    ea_out.at[cid, pl.ds(sid * ROWS_T, ROWS_T)])


_sc_ea = pl.kernel(
    _ea_body,
    out_type=[_sds((NC, N2, ED), _f32)],
    mesh=_mesh,
    scratch_types=[
        pltpu.VMEM((NBP, BLK), jnp.int32),
        pltpu.VMEM((NSLOT, BLK, ED), _f32),
        pltpu.VMEM_SHARED((N2, ED), _f32),
        pltpu.SemaphoreType.DMA((NSLOT,)),
        pltpu.SemaphoreType.DMA((NSLOT,)),
    ],
    compiler_params=_sc_params,
)


def _tc0_body(x_ref, wn_ref, ws_ref, g_ref, xs_ref):
    x = x_ref[...]
    g_ref[...] = jnp.dot(x, wn_ref[...], preferred_element_type=_f32)
    xs_ref[...] = jnp.dot(x, ws_ref[...], preferred_element_type=_f32)


def _node_update(sp_ref, eap_ref, xs_ref, we_ref, b_ref, gm_ref, bt_ref):
    s = (sp_ref[0] + sp_ref[1])[:N]
    ea = (eap_ref[0] + eap_ref[1])[:N]
    z = (s + jnp.dot(ea, we_ref[...], preferred_element_type=_f32)
         + xs_ref[...] + b_ref[...])
    u = jnp.maximum(z, 0.0)
    mean = jnp.mean(u, axis=0, keepdims=True)
    var = jnp.mean((u - mean) ** 2, axis=0, keepdims=True)
    return gm_ref[...] * (u - mean) * lax.rsqrt(var + EPS) + bt_ref[...]


def _tc_mid_body(sp_ref, eap_ref, xs_ref, we_ref, b_ref, gm_ref, bt_ref,
                 wnn_ref, wsn_ref, g_ref, xsn_ref):
    h = _node_update(sp_ref, eap_ref, xs_ref, we_ref, b_ref, gm_ref, bt_ref)
    g_ref[...] = jnp.dot(h, wnn_ref[...], preferred_element_type=_f32)
    xsn_ref[...] = jnp.dot(h, wsn_ref[...], preferred_element_type=_f32)


def _tc3_body(sp_ref, eap_ref, xs_ref, we_ref, b_ref, gm_ref, bt_ref,
              batch_ref, w1_ref, b1_ref, w2_ref, b2_ref, wo_ref, bo_ref,
              y_ref, emb_ref):
    h = _node_update(sp_ref, eap_ref, xs_ref, we_ref, b_ref, gm_ref, bt_ref)
    ids = batch_ref[...]
    neg = jnp.float32(-jnp.inf)
    seg_iota = lax.broadcasted_iota(jnp.int32, (NSEG, 64), 0)

    def seg_step(b, emb):
        v = jnp.max(jnp.where(ids == b, h, neg), axis=0, keepdims=True)
        return jnp.where(seg_iota == b, v, emb)

    embed = lax.fori_loop(0, NSEG, seg_step,
                          jnp.full((NSEG, 64), neg))
    y = jnp.maximum(jnp.dot(embed, w1_ref[...], preferred_element_type=_f32)
                    + b1_ref[...], 0.0)
    y = jnp.maximum(jnp.dot(y, w2_ref[...], preferred_element_type=_f32)
                    + b2_ref[...], 0.0)
    y = jnp.dot(y, wo_ref[...], preferred_element_type=_f32) + bo_ref[...]
    y_ref[...] = y
    emb_ref[...] = embed


_tc0 = pl.pallas_call(
    _tc0_body, out_shape=[_sds((N, 64), _f32), _sds((N, 64), _f32)])

_tc_mid = pl.pallas_call(
    _tc_mid_body, out_shape=[_sds((N, 64), _f32), _sds((N, 64), _f32)])

_tc3 = pl.pallas_call(
    _tc3_body, out_shape=[_sds((NSEG, 1), _f32), _sds((NSEG, 64), _f32)])


def _row(v):
    return v.reshape(1, -1)


def kernel(x, edge_index, edge_attr, batch, params):
    pad = NPAD * BLK - E
    srcp = jnp.concatenate(
        [edge_index[0], jnp.zeros((pad,), jnp.int32)]).reshape(NW, NBP, BLK)
    dstp = jnp.concatenate(
        [edge_index[1], jnp.full((pad,), N, jnp.int32)]).reshape(NW, NBP, BLK)
    ea3d = edge_attr.reshape(NBLK, BLK, ED)
    z64 = jnp.zeros((N2, 64), _f32)
    z16 = jnp.zeros((N2, ED), _f32)
    batch2d = batch.reshape(N, 1)

    convs = params["convs"]
    bns = params["bns"]
    lins = params["lins"]
    out_p = params["out"]

    (eap,) = _sc_ea(ea3d, dstp, z16)
    g, xs = _tc0(x, convs[0]["W_nbr"], convs[0]["W_self"])
    (sp,) = _sc_seg_sum(g, srcp, dstp, z64)

    for l in (0, 1):
        g, xs = _tc_mid(sp, eap, xs,
                        convs[l]["W_edge"], _row(convs[l]["b"]),
                        _row(bns[l]["gamma"]), _row(bns[l]["beta"]),
                        convs[l + 1]["W_nbr"], convs[l + 1]["W_self"])
        (sp,) = _sc_seg_sum(g, srcp, dstp, z64)

    y, embed = _tc3(sp, eap, xs,
                    convs[2]["W_edge"], _row(convs[2]["b"]),
                    _row(bns[2]["gamma"]), _row(bns[2]["beta"]),
                    batch2d,
                    lins[0]["W"], _row(lins[0]["b"]),
                    lins[1]["W"], _row(lins[1]["b"]),
                    out_p["W"], _row(out_p["b"]))
    return (y, embed)


# speedup vs baseline: 10.0803x; 2.1952x over previous
import jax
import jax.numpy as jnp
from jax import lax
from jax.experimental import pallas as pl
from jax.experimental.pallas import tpu as pltpu
from jax.experimental.pallas import tpu_sc as plsc

N = 10000
E = 320000
D_IN = 128
ED = 16
NSEG = 64
EPS = 1e-5

BLK = 128
NBLK = E // BLK
NC, NS = 2, 16
NW = NC * NS
NBP = 80
NPAD = NW * NBP
N2 = 10240
NSLOT = 8
DEPTH = 4
ROWS_T = N2 // NS

_f32 = jnp.float32
_mesh = plsc.VectorSubcoreMesh(core_axis_name="c", subcore_axis_name="s")
_sc_params = pltpu.CompilerParams(use_tc_tiling_on_sc=False)
_sds = jax.ShapeDtypeStruct


def _seg_sum_body(g_hbm, src_hbm, dst_hbm, z_hbm, s_out,
                  sidx, didx, rows, acc, gsem, ssem):
    cid = lax.axis_index("c")
    sid = lax.axis_index("s")
    w = sid * NC + cid

    pltpu.sync_copy(z_hbm.at[pl.ds(sid * ROWS_T, ROWS_T)],
                    acc.at[pl.ds(sid * ROWS_T, ROWS_T)])
    plsc.subcore_barrier()

    pltpu.sync_copy(src_hbm.at[w], sidx)
    pltpu.sync_copy(dst_hbm.at[w], didx)

    def real(j):
        return w * NBP + j < NBLK

    def fire_gather(j, s):
        @pl.when(real(j))
        def _():
            pltpu.async_copy(g_hbm.at[sidx.at[j]], rows.at[s], gsem.at[s])

    for s in range(DEPTH):
        fire_gather(jnp.int32(s), s)

    def it(j, carry):
        s = lax.rem(j, NSLOT)

        @pl.when(real(j))
        def _():
            pltpu.make_async_copy(g_hbm.at[sidx.at[j]], rows.at[s],
                                  gsem.at[s]).wait()
            pltpu.async_copy(rows.at[s], acc.at[didx.at[j]], ssem.at[s],
                             add=True)
        jn = j + DEPTH
        sn = lax.rem(jn, NSLOT)

        @pl.when(jn < NBP)
        def _():
            jo = jn - NSLOT

            @pl.when(jnp.logical_and(jo >= 0, real(jo)))
            def _():
                pltpu.make_async_copy(rows.at[sn], acc.at[didx.at[jo]],
                                      ssem.at[sn]).wait()
            fire_gather(jn, sn)
        return carry

    lax.fori_loop(0, NBP, it, 0)

    for s in range(NSLOT):
        @pl.when(real(jnp.int32(NBP - NSLOT + s)))
        def _():
            pltpu.make_async_copy(rows.at[s], acc.at[didx.at[0]],
                                  ssem.at[s]).wait()

    plsc.subcore_barrier()
    pltpu.sync_copy(acc.at[pl.ds(sid * ROWS_T, ROWS_T)],
                    s_out.at[cid, pl.ds(sid * ROWS_T, ROWS_T)])


_sc_seg_sum = pl.kernel(
    _seg_sum_body,
    out_type=[_sds((NC, N2, 64), _f32)],
    mesh=_mesh,
    scratch_types=[
        pltpu.VMEM((NBP, BLK), jnp.int32),
        pltpu.VMEM((NBP, BLK), jnp.int32),
        pltpu.VMEM((NSLOT, BLK, 64), _f32),
        pltpu.VMEM_SHARED((N2, 64), _f32),
        pltpu.SemaphoreType.DMA((NSLOT,)),
        pltpu.SemaphoreType.DMA((NSLOT,)),
    ],
    compiler_params=_sc_params,
)


def _ea_body(ea_hbm, dst_hbm, z_hbm, ea_out, didx, rows, acc, gsem, ssem):
    cid = lax.axis_index("c")
    sid = lax.axis_index("s")
    w = sid * NC + cid

    pltpu.sync_copy(z_hbm.at[pl.ds(sid * ROWS_T, ROWS_T)],
                    acc.at[pl.ds(sid * ROWS_T, ROWS_T)])
    plsc.subcore_barrier()

    pltpu.sync_copy(dst_hbm.at[w], didx)

    def real(j):
        return w * NBP + j < NBLK

    def fire_gather(j, s):
        @pl.when(real(j))
        def _():
            pltpu.async_copy(ea_hbm.at[w * NBP + j], rows.at[s], gsem.at[s])

    for s in range(DEPTH):
        fire_gather(jnp.int32(s), s)

    def it(j, carry):
        s = lax.rem(j, NSLOT)

        @pl.when(real(j))
        def _():
            pltpu.make_async_copy(ea_hbm.at[w * NBP + j], rows.at[s],
                                  gsem.at[s]).wait()
            pltpu.async_copy(rows.at[s], acc.at[didx.at[j]], ssem.at[s],
                             add=True)
        jn = j + DEPTH
        sn = lax.rem(jn, NSLOT)

        @pl.when(jn < NBP)
        def _():
            jo = jn - NSLOT

            @pl.when(jnp.logical_and(jo >= 0, real(jo)))
            def _():
                pltpu.make_async_copy(rows.at[sn], acc.at[didx.at[jo]],
                                      ssem.at[sn]).wait()
            fire_gather(jn, sn)
        return carry

    lax.fori_loop(0, NBP, it, 0)

    for s in range(NSLOT):
        @pl.when(real(jnp.int32(NBP - NSLOT + s)))
        def _():
            pltpu.make_async_copy(rows.at[s], acc.at[didx.at[0]],
                                  ssem.at[s]).wait()

    plsc.subcore_barrier()
    pltpu.sync_copy(acc.at[pl.ds(sid * ROWS_T, ROWS_T)],
                    ea_out.at[cid, pl.ds(sid * ROWS_T, ROWS_T)])


_sc_ea = pl.kernel(
    _ea_body,
    out_type=[_sds((NC, N2, ED), _f32)],
    mesh=_mesh,
    scratch_types=[
        pltpu.VMEM((NBP, BLK), jnp.int32),
        pltpu.VMEM((NSLOT, BLK, ED), _f32),
        pltpu.VMEM_SHARED((N2, ED), _f32),
        pltpu.SemaphoreType.DMA((NSLOT,)),
        pltpu.SemaphoreType.DMA((NSLOT,)),
    ],
    compiler_params=_sc_params,
)


def _tc0_body(x_ref, wn_ref, ws_ref, g_ref, xs_ref):
    x = x_ref[...]
    g_ref[...] = jnp.dot(x, wn_ref[...], preferred_element_type=_f32)
    xs_ref[...] = jnp.dot(x, ws_ref[...], preferred_element_type=_f32)


def _node_update(sp_ref, eap_ref, xs_ref, we_ref, b_ref, gm_ref, bt_ref):
    s = (sp_ref[0] + sp_ref[1])[:N]
    ea = (eap_ref[0] + eap_ref[1])[:N]
    z = (s + jnp.dot(ea, we_ref[...], preferred_element_type=_f32)
         + xs_ref[...] + b_ref[...])
    u = jnp.maximum(z, 0.0)
    mean = jnp.mean(u, axis=0, keepdims=True)
    var = jnp.mean((u - mean) ** 2, axis=0, keepdims=True)
    return gm_ref[...] * (u - mean) * lax.rsqrt(var + EPS) + bt_ref[...]


def _tc_mid_body(sp_ref, eap_ref, xs_ref, we_ref, b_ref, gm_ref, bt_ref,
                 wnn_ref, wsn_ref, g_ref, xsn_ref):
    h = _node_update(sp_ref, eap_ref, xs_ref, we_ref, b_ref, gm_ref, bt_ref)
    g_ref[...] = jnp.dot(h, wnn_ref[...], preferred_element_type=_f32)
    xsn_ref[...] = jnp.dot(h, wsn_ref[...], preferred_element_type=_f32)


def _tc3_body(sp_ref, eap_ref, xs_ref, we_ref, b_ref, gm_ref, bt_ref,
              batch_ref, w1_ref, b1_ref, w2_ref, b2_ref, wo_ref, bo_ref,
              y_ref, emb_ref):
    h = _node_update(sp_ref, eap_ref, xs_ref, we_ref, b_ref, gm_ref, bt_ref)
    ids = batch_ref[...]
    neg = jnp.float32(-jnp.inf)
    seg_iota = lax.broadcasted_iota(jnp.int32, (NSEG, 64), 0)

    def seg_step(b, emb):
        v = jnp.max(jnp.where(ids == b, h, neg), axis=0, keepdims=True)
        return jnp.where(seg_iota == b, v, emb)

    embed = lax.fori_loop(0, NSEG, seg_step,
                          jnp.full((NSEG, 64), neg))
    y = jnp.maximum(jnp.dot(embed, w1_ref[...], preferred_element_type=_f32)
                    + b1_ref[...], 0.0)
    y = jnp.maximum(jnp.dot(y, w2_ref[...], preferred_element_type=_f32)
                    + b2_ref[...], 0.0)
    y = jnp.dot(y, wo_ref[...], preferred_element_type=_f32) + bo_ref[...]
    y_ref[...] = y
    emb_ref[...] = embed


_tc0 = pl.pallas_call(
    _tc0_body, out_shape=[_sds((N, 64), _f32), _sds((N, 64), _f32)])

_tc_mid = pl.pallas_call(
    _tc_mid_body, out_shape=[_sds((N, 64), _f32), _sds((N, 64), _f32)])

_tc3 = pl.pallas_call(
    _tc3_body, out_shape=[_sds((NSEG, 1), _f32), _sds((NSEG, 64), _f32)])


def _row(v):
    return v.reshape(1, -1)


def kernel(x, edge_index, edge_attr, batch, params):
    pad = NPAD * BLK - E
    srcp = jnp.concatenate(
        [edge_index[0], jnp.zeros((pad,), jnp.int32)]).reshape(NW, NBP, BLK)
    dstp = jnp.concatenate(
        [edge_index[1], jnp.full((pad,), N, jnp.int32)]).reshape(NW, NBP, BLK)
    ea3d = edge_attr.reshape(NBLK, BLK, ED)
    z64 = jnp.zeros((N2, 64), _f32)
    z16 = jnp.zeros((N2, ED), _f32)
    batch2d = batch.reshape(N, 1)

    convs = params["convs"]
    bns = params["bns"]
    lins = params["lins"]
    out_p = params["out"]

    (eap,) = _sc_ea(ea3d, dstp, z16)
    g, xs = _tc0(x, convs[0]["W_nbr"], convs[0]["W_self"])
    (sp,) = _sc_seg_sum(g, srcp, dstp, z64)

    for l in (0, 1):
        g, xs = _tc_mid(sp, eap, xs,
                        convs[l]["W_edge"], _row(convs[l]["b"]),
                        _row(bns[l]["gamma"]), _row(bns[l]["beta"]),
                        convs[l + 1]["W_nbr"], convs[l + 1]["W_self"])
        (sp,) = _sc_seg_sum(g, srcp, dstp, z64)

    y, embed = _tc3(sp, eap, xs,
                    convs[2]["W_edge"], _row(convs[2]["b"]),
                    _row(bns[2]["gamma"]), _row(bns[2]["beta"]),
                    batch2d,
                    lins[0]["W"], _row(lins[0]["b"]),
                    lins[1]["W"], _row(lins[1]["b"]),
                    out_p["W"], _row(out_p["b"]))
    return (y, embed)


# speedup vs baseline: 10.4069x; 1.0324x over previous
import jax
import jax.numpy as jnp
from jax import lax
from jax.experimental import pallas as pl
from jax.experimental.pallas import tpu as pltpu
from jax.experimental.pallas import tpu_sc as plsc

N = 10000
E = 320000
D_IN = 128
ED = 16
NSEG = 64
EPS = 1e-5

BLK = 128
NBLK = E // BLK
NC, NS = 2, 16
NW = NC * NS
NBP = 80
NPAD = NW * NBP
N2 = 10240
NBLK_TAIL = NBLK - (NW - 1) * NBP
NSLOT = 8
DEPTH = 4
ROWS_T = N2 // NS
GRP = 4
NGRP = NBLK // GRP
NGRP_W = NBP // GRP

_f32 = jnp.float32
_mesh = plsc.VectorSubcoreMesh(core_axis_name="c", subcore_axis_name="s")
_sc_params = pltpu.CompilerParams(use_tc_tiling_on_sc=False)
_sds = jax.ShapeDtypeStruct


def _seg_sum_body(g_hbm, ei_hbm, z_hbm, s_out,
                  sidx, didx, rows, acc, gsem, ssem):
    cid = lax.axis_index("c")
    sid = lax.axis_index("s")
    w = sid * NC + cid

    pltpu.sync_copy(z_hbm.at[pl.ds(sid * ROWS_T, ROWS_T)],
                    acc.at[pl.ds(sid * ROWS_T, ROWS_T)])
    plsc.subcore_barrier()

    @pl.when(w < NW - 1)
    def _():
        pltpu.sync_copy(ei_hbm.at[0, pl.ds(w * NBP, NBP)], sidx)
        pltpu.sync_copy(ei_hbm.at[1, pl.ds(w * NBP, NBP)], didx)

    @pl.when(w == NW - 1)
    def _():
        pltpu.sync_copy(ei_hbm.at[0, pl.ds((NW - 1) * NBP, NBLK_TAIL)],
                        sidx.at[pl.ds(0, NBLK_TAIL)])
        pltpu.sync_copy(ei_hbm.at[1, pl.ds((NW - 1) * NBP, NBLK_TAIL)],
                        didx.at[pl.ds(0, NBLK_TAIL)])

    def real(j):
        return w * NBP + j < NBLK

    def fire_gather(j, s):
        @pl.when(real(j))
        def _():
            pltpu.async_copy(g_hbm.at[sidx.at[j]], rows.at[s], gsem.at[s])

    for s in range(DEPTH):
        fire_gather(jnp.int32(s), s)

    def it(j, carry):
        s = lax.rem(j, NSLOT)

        @pl.when(real(j))
        def _():
            pltpu.make_async_copy(g_hbm.at[sidx.at[j]], rows.at[s],
                                  gsem.at[s]).wait()
            pltpu.async_copy(rows.at[s], acc.at[didx.at[j]], ssem.at[s],
                             add=True)
        jn = j + DEPTH
        sn = lax.rem(jn, NSLOT)

        @pl.when(jn < NBP)
        def _():
            jo = jn - NSLOT

            @pl.when(jnp.logical_and(jo >= 0, real(jo)))
            def _():
                pltpu.make_async_copy(rows.at[sn], acc.at[didx.at[jo]],
                                      ssem.at[sn]).wait()
            fire_gather(jn, sn)
        return carry

    lax.fori_loop(0, NBP, it, 0)

    for s in range(NSLOT):
        @pl.when(real(jnp.int32(NBP - NSLOT + s)))
        def _():
            pltpu.make_async_copy(rows.at[s], acc.at[didx.at[0]],
                                  ssem.at[s]).wait()

    plsc.subcore_barrier()
    pltpu.sync_copy(acc.at[pl.ds(sid * ROWS_T, ROWS_T)],
                    s_out.at[cid, pl.ds(sid * ROWS_T, ROWS_T)])


_sc_seg_sum = pl.kernel(
    _seg_sum_body,
    out_type=[_sds((NC, N2, 64), _f32)],
    mesh=_mesh,
    scratch_types=[
        pltpu.VMEM((NBP, BLK), jnp.int32),
        pltpu.VMEM((NBP, BLK), jnp.int32),
        pltpu.VMEM((NSLOT, BLK, 64), _f32),
        pltpu.VMEM_SHARED((N2, 64), _f32),
        pltpu.SemaphoreType.DMA((NSLOT,)),
        pltpu.SemaphoreType.DMA((NSLOT,)),
    ],
    compiler_params=_sc_params,
)


def _ea_body(ea_hbm, ei_hbm, z_hbm, ea_out, didx, slots, acc, gsem, ssem):
    cid = lax.axis_index("c")
    sid = lax.axis_index("s")
    w = sid * NC + cid

    pltpu.sync_copy(z_hbm.at[pl.ds(sid * ROWS_T, ROWS_T)],
                    acc.at[pl.ds(sid * ROWS_T, ROWS_T)])
    plsc.subcore_barrier()

    @pl.when(w < NW - 1)
    def _():
        pltpu.sync_copy(ei_hbm.at[1, pl.ds(w * NBP, NBP)], didx)

    @pl.when(w == NW - 1)
    def _():
        pltpu.sync_copy(ei_hbm.at[1, pl.ds((NW - 1) * NBP, NBLK_TAIL)],
                        didx.at[pl.ds(0, NBLK_TAIL)])

    def real(g):
        return w * NGRP_W + g < NGRP

    def fire_gather(g, s):
        @pl.when(real(g))
        def _():
            pltpu.async_copy(ea_hbm.at[w * NGRP_W + g], slots.at[s],
                             gsem.at[s])

    for s in range(DEPTH):
        fire_gather(jnp.int32(s), s)

    def wait_scat(s):
        for _q in range(GRP):
            pltpu.make_async_copy(slots.at[s, 0], acc.at[didx.at[0]],
                                  ssem.at[s]).wait()

    def it(g, carry):
        s = lax.rem(g, NSLOT)

        @pl.when(real(g))
        def _():
            pltpu.make_async_copy(ea_hbm.at[w * NGRP_W + g], slots.at[s],
                                  gsem.at[s]).wait()
            for q in range(GRP):
                pltpu.async_copy(slots.at[s, q], acc.at[didx.at[g * GRP + q]],
                                 ssem.at[s], add=True)
        gn = g + DEPTH
        sn = lax.rem(gn, NSLOT)

        @pl.when(gn < NGRP_W)
        def _():
            go = gn - NSLOT

            @pl.when(jnp.logical_and(go >= 0, real(go)))
            def _():
                wait_scat(sn)
            fire_gather(gn, sn)
        return carry

    lax.fori_loop(0, NGRP_W, it, 0)

    for s in range(NSLOT):
        @pl.when(real(jnp.int32(NGRP_W - NSLOT + s)))
        def _():
            wait_scat(s)

    plsc.subcore_barrier()
    pltpu.sync_copy(acc.at[pl.ds(sid * ROWS_T, ROWS_T)],
                    ea_out.at[cid, pl.ds(sid * ROWS_T, ROWS_T)])


_sc_ea = pl.kernel(
    _ea_body,
    out_type=[_sds((NC, N2, ED), _f32)],
    mesh=_mesh,
    scratch_types=[
        pltpu.VMEM((NBP, BLK), jnp.int32),
        pltpu.VMEM((NSLOT, GRP, BLK, ED), _f32),
        pltpu.VMEM_SHARED((N2, ED), _f32),
        pltpu.SemaphoreType.DMA((NSLOT,)),
        pltpu.SemaphoreType.DMA((NSLOT,)),
    ],
    compiler_params=_sc_params,
)


def _tc0_body(x_ref, wn_ref, ws_ref, g_ref, xs_ref):
    x = x_ref[...]
    g_ref[...] = jnp.dot(x, wn_ref[...], preferred_element_type=_f32)
    xs_ref[...] = jnp.dot(x, ws_ref[...], preferred_element_type=_f32)


def _node_update(sp_ref, eap_ref, xs_ref, we_ref, b_ref, gm_ref, bt_ref):
    s = (sp_ref[0] + sp_ref[1])[:N]
    ea = (eap_ref[0] + eap_ref[1])[:N]
    z = (s + jnp.dot(ea, we_ref[...], preferred_element_type=_f32)
         + xs_ref[...] + b_ref[...])
    u = jnp.maximum(z, 0.0)
    mean = jnp.mean(u, axis=0, keepdims=True)
    var = jnp.mean((u - mean) ** 2, axis=0, keepdims=True)
    return gm_ref[...] * (u - mean) * lax.rsqrt(var + EPS) + bt_ref[...]


def _tc_mid_body(sp_ref, eap_ref, xs_ref, we_ref, b_ref, gm_ref, bt_ref,
                 wnn_ref, wsn_ref, g_ref, xsn_ref):
    h = _node_update(sp_ref, eap_ref, xs_ref, we_ref, b_ref, gm_ref, bt_ref)
    g_ref[...] = jnp.dot(h, wnn_ref[...], preferred_element_type=_f32)
    xsn_ref[...] = jnp.dot(h, wsn_ref[...], preferred_element_type=_f32)


def _tc3_body(sp_ref, eap_ref, xs_ref, we_ref, b_ref, gm_ref, bt_ref,
              batch_ref, w1_ref, b1_ref, w2_ref, b2_ref, wo_ref, bo_ref,
              y_ref, emb_ref):
    h = _node_update(sp_ref, eap_ref, xs_ref, we_ref, b_ref, gm_ref, bt_ref)
    ids = batch_ref[...]
    neg = jnp.float32(-jnp.inf)
    seg_iota = lax.broadcasted_iota(jnp.int32, (NSEG, 64), 0)

    def seg_step(b, emb):
        v = jnp.max(jnp.where(ids == b, h, neg), axis=0, keepdims=True)
        return jnp.where(seg_iota == b, v, emb)

    embed = lax.fori_loop(0, NSEG, seg_step,
                          jnp.full((NSEG, 64), neg))
    y = jnp.maximum(jnp.dot(embed, w1_ref[...], preferred_element_type=_f32)
                    + b1_ref[...], 0.0)
    y = jnp.maximum(jnp.dot(y, w2_ref[...], preferred_element_type=_f32)
                    + b2_ref[...], 0.0)
    y = jnp.dot(y, wo_ref[...], preferred_element_type=_f32) + bo_ref[...]
    y_ref[...] = y
    emb_ref[...] = embed


_tc0 = pl.pallas_call(
    _tc0_body, out_shape=[_sds((N, 64), _f32), _sds((N, 64), _f32)])

_tc_mid = pl.pallas_call(
    _tc_mid_body, out_shape=[_sds((N, 64), _f32), _sds((N, 64), _f32)])

_tc3 = pl.pallas_call(
    _tc3_body, out_shape=[_sds((NSEG, 1), _f32), _sds((NSEG, 64), _f32)])


def _row(v):
    return v.reshape(1, -1)


def kernel(x, edge_index, edge_attr, batch, params):
    ei3 = edge_index.reshape(2, NBLK, BLK)
    ea4 = edge_attr.reshape(NGRP, GRP, BLK, ED)
    z64 = jnp.zeros((N2, 64), _f32)
    z16 = jnp.zeros((N2, ED), _f32)
    batch2d = batch.reshape(N, 1)

    convs = params["convs"]
    bns = params["bns"]
    lins = params["lins"]
    out_p = params["out"]

    (eap,) = _sc_ea(ea4, ei3, z16)
    g, xs = _tc0(x, convs[0]["W_nbr"], convs[0]["W_self"])
    (sp,) = _sc_seg_sum(g, ei3, z64)

    for l in (0, 1):
        g, xs = _tc_mid(sp, eap, xs,
                        convs[l]["W_edge"], _row(convs[l]["b"]),
                        _row(bns[l]["gamma"]), _row(bns[l]["beta"]),
                        convs[l + 1]["W_nbr"], convs[l + 1]["W_self"])
        (sp,) = _sc_seg_sum(g, ei3, z64)

    y, embed = _tc3(sp, eap, xs,
                    convs[2]["W_edge"], _row(convs[2]["b"]),
                    _row(bns[2]["gamma"]), _row(bns[2]["beta"]),
                    batch2d,
                    lins[0]["W"], _row(lins[0]["b"]),
                    lins[1]["W"], _row(lins[1]["b"]),
                    out_p["W"], _row(out_p["b"]))
    return (y, embed)


# speedup vs baseline: 12.2000x; 1.1723x over previous
import jax
import jax.numpy as jnp
from jax import lax
from jax.experimental import pallas as pl
from jax.experimental.pallas import tpu as pltpu
from jax.experimental.pallas import tpu_sc as plsc

N = 10000
E = 320000
D_IN = 128
ED = 16
NSEG = 64
EPS = 1e-5

BLK = 128
NBLK = E // BLK
NC, NS = 2, 16
NW = NC * NS
NBP = 80
NPAD = NW * NBP
N2 = 10240
NBLK_TAIL = NBLK - (NW - 1) * NBP
NSLOT = 8
DEPTH = 4
ROWS_T = N2 // NS
GRP = 4
NGRP = NBLK // GRP
NGRP_W = NBP // GRP

_f32 = jnp.float32
_mesh = plsc.VectorSubcoreMesh(core_axis_name="c", subcore_axis_name="s")
_sc_params = pltpu.CompilerParams(use_tc_tiling_on_sc=False)
_sds = jax.ShapeDtypeStruct


def _seg_sum_body(g_hbm, ei_hbm, z_hbm, s_out,
                  sidx, didx, rows, acc, gsem, ssem):
    cid = lax.axis_index("c")
    sid = lax.axis_index("s")
    w = sid * NC + cid

    pltpu.sync_copy(z_hbm.at[pl.ds(sid * ROWS_T, ROWS_T)],
                    acc.at[pl.ds(sid * ROWS_T, ROWS_T)])
    plsc.subcore_barrier()

    @pl.when(w < NW - 1)
    def _():
        pltpu.sync_copy(ei_hbm.at[0, pl.ds(w * NBP, NBP)], sidx)
        pltpu.sync_copy(ei_hbm.at[1, pl.ds(w * NBP, NBP)], didx)

    @pl.when(w == NW - 1)
    def _():
        pltpu.sync_copy(ei_hbm.at[0, pl.ds((NW - 1) * NBP, NBLK_TAIL)],
                        sidx.at[pl.ds(0, NBLK_TAIL)])
        pltpu.sync_copy(ei_hbm.at[1, pl.ds((NW - 1) * NBP, NBLK_TAIL)],
                        didx.at[pl.ds(0, NBLK_TAIL)])

    def real(j):
        return w * NBP + j < NBLK

    def fire_gather(j, s):
        @pl.when(real(j))
        def _():
            pltpu.async_copy(g_hbm.at[sidx.at[j]], rows.at[s], gsem.at[s])

    for s in range(DEPTH):
        fire_gather(jnp.int32(s), s)

    def it(j, carry):
        s = lax.rem(j, NSLOT)

        @pl.when(real(j))
        def _():
            pltpu.make_async_copy(g_hbm.at[sidx.at[j]], rows.at[s],
                                  gsem.at[s]).wait()
            pltpu.async_copy(rows.at[s], acc.at[didx.at[j]], ssem.at[s],
                             add=True)
        jn = j + DEPTH
        sn = lax.rem(jn, NSLOT)

        @pl.when(jn < NBP)
        def _():
            jo = jn - NSLOT

            @pl.when(jnp.logical_and(jo >= 0, real(jo)))
            def _():
                pltpu.make_async_copy(rows.at[sn], acc.at[didx.at[jo]],
                                      ssem.at[sn]).wait()
            fire_gather(jn, sn)
        return carry

    lax.fori_loop(0, NBP, it, 0)

    for s in range(NSLOT):
        @pl.when(real(jnp.int32(NBP - NSLOT + s)))
        def _():
            pltpu.make_async_copy(rows.at[s], acc.at[didx.at[0]],
                                  ssem.at[s]).wait()

    plsc.subcore_barrier()
    pltpu.sync_copy(acc.at[pl.ds(sid * ROWS_T, ROWS_T)],
                    s_out.at[cid, pl.ds(sid * ROWS_T, ROWS_T)])


_sc_seg_sum = pl.kernel(
    _seg_sum_body,
    out_type=[_sds((NC, N2, 64), _f32)],
    mesh=_mesh,
    scratch_types=[
        pltpu.VMEM((NBP, BLK), jnp.int32),
        pltpu.VMEM((NBP, BLK), jnp.int32),
        pltpu.VMEM((NSLOT, BLK, 64), _f32),
        pltpu.VMEM_SHARED((N2, 64), _f32),
        pltpu.SemaphoreType.DMA((NSLOT,)),
        pltpu.SemaphoreType.DMA((NSLOT,)),
    ],
    compiler_params=_sc_params,
)


def _ea_body(ea_hbm, ei_hbm, z_hbm, ea_out, didx, slots, acc, gsem, ssem):
    cid = lax.axis_index("c")
    sid = lax.axis_index("s")
    w = sid * NC + cid

    pltpu.sync_copy(z_hbm.at[pl.ds(sid * ROWS_T, ROWS_T)],
                    acc.at[pl.ds(sid * ROWS_T, ROWS_T)])
    plsc.subcore_barrier()

    @pl.when(w < NW - 1)
    def _():
        pltpu.sync_copy(ei_hbm.at[1, pl.ds(w * NBP, NBP)], didx)

    @pl.when(w == NW - 1)
    def _():
        pltpu.sync_copy(ei_hbm.at[1, pl.ds((NW - 1) * NBP, NBLK_TAIL)],
                        didx.at[pl.ds(0, NBLK_TAIL)])

    def real(g):
        return w * NGRP_W + g < NGRP

    def fire_gather(g, s):
        @pl.when(real(g))
        def _():
            pltpu.async_copy(ea_hbm.at[w * NGRP_W + g], slots.at[s],
                             gsem.at[s])

    for s in range(DEPTH):
        fire_gather(jnp.int32(s), s)

    def wait_scat(s):
        for _q in range(GRP):
            pltpu.make_async_copy(slots.at[s, 0], acc.at[didx.at[0]],
                                  ssem.at[s]).wait()

    def it(g, carry):
        s = lax.rem(g, NSLOT)

        @pl.when(real(g))
        def _():
            pltpu.make_async_copy(ea_hbm.at[w * NGRP_W + g], slots.at[s],
                                  gsem.at[s]).wait()
            for q in range(GRP):
                pltpu.async_copy(slots.at[s, q], acc.at[didx.at[g * GRP + q]],
                                 ssem.at[s], add=True)
        gn = g + DEPTH
        sn = lax.rem(gn, NSLOT)

        @pl.when(gn < NGRP_W)
        def _():
            go = gn - NSLOT

            @pl.when(jnp.logical_and(go >= 0, real(go)))
            def _():
                wait_scat(sn)
            fire_gather(gn, sn)
        return carry

    lax.fori_loop(0, NGRP_W, it, 0)

    for s in range(NSLOT):
        @pl.when(real(jnp.int32(NGRP_W - NSLOT + s)))
        def _():
            wait_scat(s)

    plsc.subcore_barrier()
    pltpu.sync_copy(acc.at[pl.ds(sid * ROWS_T, ROWS_T)],
                    ea_out.at[cid, pl.ds(sid * ROWS_T, ROWS_T)])


_sc_ea = pl.kernel(
    _ea_body,
    out_type=[_sds((NC, N2, ED), _f32)],
    mesh=_mesh,
    scratch_types=[
        pltpu.VMEM((NBP, BLK), jnp.int32),
        pltpu.VMEM((NSLOT, GRP, BLK, ED), _f32),
        pltpu.VMEM_SHARED((N2, ED), _f32),
        pltpu.SemaphoreType.DMA((NSLOT,)),
        pltpu.SemaphoreType.DMA((NSLOT,)),
    ],
    compiler_params=_sc_params,
)


def _pool_body(h_hbm, b_hbm, p_out, rows, bids, acc):
    cid = lax.axis_index("c")
    sid = lax.axis_index("s")
    w = sid * NC + cid
    RPW = N2 // NW
    base = w * RPW

    pltpu.sync_copy(h_hbm.at[pl.ds(base, RPW)], rows)
    pltpu.sync_copy(b_hbm.at[pl.ds(base, RPW)], bids)

    neg = jnp.full((16,), -jnp.inf, _f32)

    def zr(i, carry):
        for q in range(4):
            acc[i, pl.ds(q * 16, 16)] = neg
        return carry

    lax.fori_loop(0, NSEG + 1, zr, 0)

    def it(g, carry):
        idv = bids[pl.ds(g * 16, 16)]
        for i in range(16):
            seg = idv[i]
            r = g * 16 + i
            for q in range(4):
                v = rows[r, pl.ds(q * 16, 16)]
                a = acc[seg, pl.ds(q * 16, 16)]
                acc[seg, pl.ds(q * 16, 16)] = jnp.maximum(a, v)
        return carry

    lax.fori_loop(0, RPW // 16, it, 0)
    pltpu.sync_copy(acc, p_out.at[w])


_sc_pool = pl.kernel(
    _pool_body,
    out_type=[_sds((NW, NSEG + 1, 64), _f32)],
    mesh=_mesh,
    scratch_types=[
        pltpu.VMEM((N2 // NW, 64), _f32),
        pltpu.VMEM((N2 // NW,), jnp.int32),
        pltpu.VMEM((NSEG + 1, 64), _f32),
    ],
    compiler_params=_sc_params,
)


def _tc0_body(x_ref, wn_ref, ws_ref, g_ref, xs_ref):
    x = x_ref[...]
    g_ref[...] = jnp.dot(x, wn_ref[...], preferred_element_type=_f32)
    xs_ref[...] = jnp.dot(x, ws_ref[...], preferred_element_type=_f32)


def _node_update(sp_ref, eap_ref, xs_ref, we_ref, b_ref, gm_ref, bt_ref):
    s = (sp_ref[0] + sp_ref[1])[:N]
    ea = (eap_ref[0] + eap_ref[1])[:N]
    z = (s + jnp.dot(ea, we_ref[...], preferred_element_type=_f32)
         + xs_ref[...] + b_ref[...])
    u = jnp.maximum(z, 0.0)
    mean = jnp.mean(u, axis=0, keepdims=True)
    var = jnp.mean((u - mean) ** 2, axis=0, keepdims=True)
    return gm_ref[...] * (u - mean) * lax.rsqrt(var + EPS) + bt_ref[...]


def _tc_mid_body(sp_ref, eap_ref, xs_ref, we_ref, b_ref, gm_ref, bt_ref,
                 wnn_ref, wsn_ref, g_ref, xsn_ref):
    h = _node_update(sp_ref, eap_ref, xs_ref, we_ref, b_ref, gm_ref, bt_ref)
    g_ref[...] = jnp.dot(h, wnn_ref[...], preferred_element_type=_f32)
    xsn_ref[...] = jnp.dot(h, wsn_ref[...], preferred_element_type=_f32)


def _tc3_body(sp_ref, eap_ref, xs_ref, we_ref, b_ref, gm_ref, bt_ref,
              h_ref):
    h = _node_update(sp_ref, eap_ref, xs_ref, we_ref, b_ref, gm_ref, bt_ref)
    h_ref[pl.ds(0, N), :] = h
    h_ref[pl.ds(N, N2 - N), :] = jnp.zeros((N2 - N, 64), _f32)


def _tc4_body(p_ref, w1_ref, b1_ref, w2_ref, b2_ref, wo_ref, bo_ref,
              y_ref, emb_ref):
    embed = jnp.max(p_ref[:, :NSEG, :], axis=0)
    y = jnp.maximum(jnp.dot(embed, w1_ref[...], preferred_element_type=_f32)
                    + b1_ref[...], 0.0)
    y = jnp.maximum(jnp.dot(y, w2_ref[...], preferred_element_type=_f32)
                    + b2_ref[...], 0.0)
    y = jnp.dot(y, wo_ref[...], preferred_element_type=_f32) + bo_ref[...]
    y_ref[...] = y
    emb_ref[...] = embed


_tc0 = pl.pallas_call(
    _tc0_body, out_shape=[_sds((N, 64), _f32), _sds((N, 64), _f32)])

_tc_mid = pl.pallas_call(
    _tc_mid_body, out_shape=[_sds((N, 64), _f32), _sds((N, 64), _f32)])

_tc3 = pl.pallas_call(
    _tc3_body, out_shape=[_sds((N2, 64), _f32)])

_tc4 = pl.pallas_call(
    _tc4_body, out_shape=[_sds((NSEG, 1), _f32), _sds((NSEG, 64), _f32)])


def _row(v):
    return v.reshape(1, -1)


def kernel(x, edge_index, edge_attr, batch, params):
    ei3 = edge_index.reshape(2, NBLK, BLK)
    ea4 = edge_attr.reshape(NGRP, GRP, BLK, ED)
    z64 = jnp.zeros((N2, 64), _f32)
    z16 = jnp.zeros((N2, ED), _f32)
    bpad = jnp.concatenate(
        [batch, jnp.full((N2 - N,), NSEG, jnp.int32)])

    convs = params["convs"]
    bns = params["bns"]
    lins = params["lins"]
    out_p = params["out"]

    (eap,) = _sc_ea(ea4, ei3, z16)
    g, xs = _tc0(x, convs[0]["W_nbr"], convs[0]["W_self"])
    (sp,) = _sc_seg_sum(g, ei3, z64)

    for l in (0, 1):
        g, xs = _tc_mid(sp, eap, xs,
                        convs[l]["W_edge"], _row(convs[l]["b"]),
                        _row(bns[l]["gamma"]), _row(bns[l]["beta"]),
                        convs[l + 1]["W_nbr"], convs[l + 1]["W_self"])
        (sp,) = _sc_seg_sum(g, ei3, z64)

    (h3,) = _tc3(sp, eap, xs,
                 convs[2]["W_edge"], _row(convs[2]["b"]),
                 _row(bns[2]["gamma"]), _row(bns[2]["beta"]))
    (parts,) = _sc_pool(h3, bpad)
    y, embed = _tc4(parts,
                    lins[0]["W"], _row(lins[0]["b"]),
                    lins[1]["W"], _row(lins[1]["b"]),
                    out_p["W"], _row(out_p["b"]))
    return (y, embed)


# speedup vs baseline: 12.4752x; 1.0226x over previous
import jax
import jax.numpy as jnp
from jax import lax
from jax.experimental import pallas as pl
from jax.experimental.pallas import tpu as pltpu
from jax.experimental.pallas import tpu_sc as plsc

N = 10000
E = 320000
D_IN = 128
ED = 16
NSEG = 64
EPS = 1e-5

BLK = 128
NBLK = E // BLK
NC, NS = 2, 16
NW = NC * NS
NBP = 80
NPAD = NW * NBP
N2 = 10240
NBLK_TAIL = NBLK - (NW - 1) * NBP
NSLOT = 8
DEPTH = 6
ROWS_T = N2 // NS
GRP = 4
NGRP = NBLK // GRP
NGRP_W = NBP // GRP

_f32 = jnp.float32
_mesh = plsc.VectorSubcoreMesh(core_axis_name="c", subcore_axis_name="s")
_sc_params = pltpu.CompilerParams(use_tc_tiling_on_sc=False)
_sds = jax.ShapeDtypeStruct


def _seg_sum_body(g_hbm, ei_hbm, z_hbm, s_out,
                  sidx, didx, rows, acc, gsem, ssem):
    cid = lax.axis_index("c")
    sid = lax.axis_index("s")
    w = sid * NC + cid

    pltpu.sync_copy(z_hbm.at[pl.ds(sid * ROWS_T, ROWS_T)],
                    acc.at[pl.ds(sid * ROWS_T, ROWS_T)])
    plsc.subcore_barrier()

    @pl.when(w < NW - 1)
    def _():
        pltpu.sync_copy(ei_hbm.at[0, pl.ds(w * NBP, NBP)], sidx)
        pltpu.sync_copy(ei_hbm.at[1, pl.ds(w * NBP, NBP)], didx)

    @pl.when(w == NW - 1)
    def _():
        pltpu.sync_copy(ei_hbm.at[0, pl.ds((NW - 1) * NBP, NBLK_TAIL)],
                        sidx.at[pl.ds(0, NBLK_TAIL)])
        pltpu.sync_copy(ei_hbm.at[1, pl.ds((NW - 1) * NBP, NBLK_TAIL)],
                        didx.at[pl.ds(0, NBLK_TAIL)])

    def real(j):
        return w * NBP + j < NBLK

    def fire_gather(j, s):
        @pl.when(real(j))
        def _():
            pltpu.async_copy(g_hbm.at[sidx.at[j]], rows.at[s], gsem.at[s])

    for s in range(DEPTH):
        fire_gather(jnp.int32(s), s)

    def it(j, carry):
        s = lax.rem(j, NSLOT)

        @pl.when(real(j))
        def _():
            pltpu.make_async_copy(g_hbm.at[sidx.at[j]], rows.at[s],
                                  gsem.at[s]).wait()
            pltpu.async_copy(rows.at[s], acc.at[didx.at[j]], ssem.at[s],
                             add=True)
        jn = j + DEPTH
        sn = lax.rem(jn, NSLOT)

        @pl.when(jn < NBP)
        def _():
            jo = jn - NSLOT

            @pl.when(jnp.logical_and(jo >= 0, real(jo)))
            def _():
                pltpu.make_async_copy(rows.at[sn], acc.at[didx.at[jo]],
                                      ssem.at[sn]).wait()
            fire_gather(jn, sn)
        return carry

    lax.fori_loop(0, NBP, it, 0)

    for s in range(NSLOT):
        @pl.when(real(jnp.int32(NBP - NSLOT + s)))
        def _():
            pltpu.make_async_copy(rows.at[s], acc.at[didx.at[0]],
                                  ssem.at[s]).wait()

    plsc.subcore_barrier()
    pltpu.sync_copy(acc.at[pl.ds(sid * ROWS_T, ROWS_T)],
                    s_out.at[cid, pl.ds(sid * ROWS_T, ROWS_T)])


_sc_seg_sum = pl.kernel(
    _seg_sum_body,
    out_type=[_sds((NC, N2, 64), _f32)],
    mesh=_mesh,
    scratch_types=[
        pltpu.VMEM((NBP, BLK), jnp.int32),
        pltpu.VMEM((NBP, BLK), jnp.int32),
        pltpu.VMEM((NSLOT, BLK, 64), _f32),
        pltpu.VMEM_SHARED((N2, 64), _f32),
        pltpu.SemaphoreType.DMA((NSLOT,)),
        pltpu.SemaphoreType.DMA((NSLOT,)),
    ],
    compiler_params=_sc_params,
)


def _ea_body(ea_hbm, ei_hbm, z_hbm, ea_out, didx, slots, acc, gsem, ssem):
    cid = lax.axis_index("c")
    sid = lax.axis_index("s")
    w = sid * NC + cid

    pltpu.sync_copy(z_hbm.at[pl.ds(sid * ROWS_T, ROWS_T)],
                    acc.at[pl.ds(sid * ROWS_T, ROWS_T)])
    plsc.subcore_barrier()

    @pl.when(w < NW - 1)
    def _():
        pltpu.sync_copy(ei_hbm.at[1, pl.ds(w * NBP, NBP)], didx)

    @pl.when(w == NW - 1)
    def _():
        pltpu.sync_copy(ei_hbm.at[1, pl.ds((NW - 1) * NBP, NBLK_TAIL)],
                        didx.at[pl.ds(0, NBLK_TAIL)])

    def real(g):
        return w * NGRP_W + g < NGRP

    def fire_gather(g, s):
        @pl.when(real(g))
        def _():
            pltpu.async_copy(ea_hbm.at[w * NGRP_W + g], slots.at[s],
                             gsem.at[s])

    for s in range(DEPTH):
        fire_gather(jnp.int32(s), s)

    def wait_scat(s):
        for _q in range(GRP):
            pltpu.make_async_copy(slots.at[s, 0], acc.at[didx.at[0]],
                                  ssem.at[s]).wait()

    def it(g, carry):
        s = lax.rem(g, NSLOT)

        @pl.when(real(g))
        def _():
            pltpu.make_async_copy(ea_hbm.at[w * NGRP_W + g], slots.at[s],
                                  gsem.at[s]).wait()
            for q in range(GRP):
                pltpu.async_copy(slots.at[s, q], acc.at[didx.at[g * GRP + q]],
                                 ssem.at[s], add=True)
        gn = g + DEPTH
        sn = lax.rem(gn, NSLOT)

        @pl.when(gn < NGRP_W)
        def _():
            go = gn - NSLOT

            @pl.when(jnp.logical_and(go >= 0, real(go)))
            def _():
                wait_scat(sn)
            fire_gather(gn, sn)
        return carry

    lax.fori_loop(0, NGRP_W, it, 0)

    for s in range(NSLOT):
        @pl.when(real(jnp.int32(NGRP_W - NSLOT + s)))
        def _():
            wait_scat(s)

    plsc.subcore_barrier()
    pltpu.sync_copy(acc.at[pl.ds(sid * ROWS_T, ROWS_T)],
                    ea_out.at[cid, pl.ds(sid * ROWS_T, ROWS_T)])


_sc_ea = pl.kernel(
    _ea_body,
    out_type=[_sds((NC, N2, ED), _f32)],
    mesh=_mesh,
    scratch_types=[
        pltpu.VMEM((NBP, BLK), jnp.int32),
        pltpu.VMEM((NSLOT, GRP, BLK, ED), _f32),
        pltpu.VMEM_SHARED((N2, ED), _f32),
        pltpu.SemaphoreType.DMA((NSLOT,)),
        pltpu.SemaphoreType.DMA((NSLOT,)),
    ],
    compiler_params=_sc_params,
)


def _pool_body(h_hbm, b_hbm, p_out, rows, bids, acc):
    cid = lax.axis_index("c")
    sid = lax.axis_index("s")
    w = sid * NC + cid
    RPW = N2 // NW
    base = w * RPW

    pltpu.sync_copy(h_hbm.at[pl.ds(base, RPW)], rows)
    pltpu.sync_copy(b_hbm.at[pl.ds(base, RPW)], bids)

    neg = jnp.full((16,), -jnp.inf, _f32)

    def zr(i, carry):
        for q in range(4):
            acc[i, pl.ds(q * 16, 16)] = neg
        return carry

    lax.fori_loop(0, NSEG + 1, zr, 0)

    def it(g, carry):
        idv = bids[pl.ds(g * 16, 16)]
        for i in range(16):
            seg = idv[i]
            r = g * 16 + i
            for q in range(4):
                v = rows[r, pl.ds(q * 16, 16)]
                a = acc[seg, pl.ds(q * 16, 16)]
                acc[seg, pl.ds(q * 16, 16)] = jnp.maximum(a, v)
        return carry

    lax.fori_loop(0, RPW // 16, it, 0)
    pltpu.sync_copy(acc, p_out.at[w])


_sc_pool = pl.kernel(
    _pool_body,
    out_type=[_sds((NW, NSEG + 1, 64), _f32)],
    mesh=_mesh,
    scratch_types=[
        pltpu.VMEM((N2 // NW, 64), _f32),
        pltpu.VMEM((N2 // NW,), jnp.int32),
        pltpu.VMEM((NSEG + 1, 64), _f32),
    ],
    compiler_params=_sc_params,
)


def _tc0_body(x_ref, wn_ref, ws_ref, g_ref, xs_ref):
    x = x_ref[...]
    g_ref[...] = jnp.dot(x, wn_ref[...], preferred_element_type=_f32)
    xs_ref[...] = jnp.dot(x, ws_ref[...], preferred_element_type=_f32)


def _node_update(sp_ref, eap_ref, xs_ref, we_ref, b_ref, gm_ref, bt_ref):
    s = (sp_ref[0] + sp_ref[1])[:N]
    ea = (eap_ref[0] + eap_ref[1])[:N]
    z = (s + jnp.dot(ea, we_ref[...], preferred_element_type=_f32)
         + xs_ref[...] + b_ref[...])
    u = jnp.maximum(z, 0.0)
    mean = jnp.mean(u, axis=0, keepdims=True)
    var = jnp.mean((u - mean) ** 2, axis=0, keepdims=True)
    return gm_ref[...] * (u - mean) * lax.rsqrt(var + EPS) + bt_ref[...]


def _tc_mid_body(sp_ref, eap_ref, xs_ref, we_ref, b_ref, gm_ref, bt_ref,
                 wnn_ref, wsn_ref, g_ref, xsn_ref):
    h = _node_update(sp_ref, eap_ref, xs_ref, we_ref, b_ref, gm_ref, bt_ref)
    g_ref[...] = jnp.dot(h, wnn_ref[...], preferred_element_type=_f32)
    xsn_ref[...] = jnp.dot(h, wsn_ref[...], preferred_element_type=_f32)


def _tc3_body(sp_ref, eap_ref, xs_ref, we_ref, b_ref, gm_ref, bt_ref,
              h_ref):
    h = _node_update(sp_ref, eap_ref, xs_ref, we_ref, b_ref, gm_ref, bt_ref)
    h_ref[pl.ds(0, N), :] = h
    h_ref[pl.ds(N, N2 - N), :] = jnp.zeros((N2 - N, 64), _f32)


def _tc4_body(p_ref, w1_ref, b1_ref, w2_ref, b2_ref, wo_ref, bo_ref,
              y_ref, emb_ref):
    embed = jnp.max(p_ref[:, :NSEG, :], axis=0)
    y = jnp.maximum(jnp.dot(embed, w1_ref[...], preferred_element_type=_f32)
                    + b1_ref[...], 0.0)
    y = jnp.maximum(jnp.dot(y, w2_ref[...], preferred_element_type=_f32)
                    + b2_ref[...], 0.0)
    y = jnp.dot(y, wo_ref[...], preferred_element_type=_f32) + bo_ref[...]
    y_ref[...] = y
    emb_ref[...] = embed


_tc0 = pl.pallas_call(
    _tc0_body, out_shape=[_sds((N, 64), _f32), _sds((N, 64), _f32)])

_tc_mid = pl.pallas_call(
    _tc_mid_body, out_shape=[_sds((N, 64), _f32), _sds((N, 64), _f32)])

_tc3 = pl.pallas_call(
    _tc3_body, out_shape=[_sds((N2, 64), _f32)])

_tc4 = pl.pallas_call(
    _tc4_body, out_shape=[_sds((NSEG, 1), _f32), _sds((NSEG, 64), _f32)])


def _row(v):
    return v.reshape(1, -1)


def kernel(x, edge_index, edge_attr, batch, params):
    ei3 = edge_index.reshape(2, NBLK, BLK)
    ea4 = edge_attr.reshape(NGRP, GRP, BLK, ED)
    z64 = jnp.zeros((N2, 64), _f32)
    z16 = jnp.zeros((N2, ED), _f32)
    bpad = jnp.concatenate(
        [batch, jnp.full((N2 - N,), NSEG, jnp.int32)])

    convs = params["convs"]
    bns = params["bns"]
    lins = params["lins"]
    out_p = params["out"]

    (eap,) = _sc_ea(ea4, ei3, z16)
    g, xs = _tc0(x, convs[0]["W_nbr"], convs[0]["W_self"])
    (sp,) = _sc_seg_sum(g, ei3, z64)

    for l in (0, 1):
        g, xs = _tc_mid(sp, eap, xs,
                        convs[l]["W_edge"], _row(convs[l]["b"]),
                        _row(bns[l]["gamma"]), _row(bns[l]["beta"]),
                        convs[l + 1]["W_nbr"], convs[l + 1]["W_self"])
        (sp,) = _sc_seg_sum(g, ei3, z64)

    (h3,) = _tc3(sp, eap, xs,
                 convs[2]["W_edge"], _row(convs[2]["b"]),
                 _row(bns[2]["gamma"]), _row(bns[2]["beta"]))
    (parts,) = _sc_pool(h3, bpad)
    y, embed = _tc4(parts,
                    lins[0]["W"], _row(lins[0]["b"]),
                    lins[1]["W"], _row(lins[1]["b"]),
                    out_p["W"], _row(out_p["b"]))
    return (y, embed)
